# baseline (speedup 1.0000x reference)
import functools

import jax
import jax.numpy as jnp
import numpy as np
from jax import lax
from jax.experimental import pallas as pl
from jax.experimental.pallas import tpu as pltpu
from jax.experimental.pallas import tpu_sc as plsc

_DIM = 2048
_BATCH = 4096
_LANES = 16
_NUM_WORKERS = 32
_ROWS_PER_WORKER = _BATCH // _NUM_WORKERS
_CHUNK = 8
_NUM_CHUNKS = _ROWS_PER_WORKER // _CHUNK
_GROUPS = _DIM // _LANES
_NBUF_IN = 4
_NBUF_OUT = 2

_PERM = np.asarray(jax.random.permutation(jax.random.key(42), _DIM),
                   dtype=np.int32)


def _conflict_free_schedule(perm: np.ndarray):
    n, lanes = perm.shape[0], _LANES
    buckets = [[[] for _ in range(lanes)] for _ in range(lanes)]
    for dst in range(n):
        buckets[int(perm[dst]) % lanes][dst % lanes].append(dst)

    match_src = [0] * lanes
    gather_idx = np.empty(n, dtype=np.int32)
    scatter_idx = np.empty(n, dtype=np.int32)
    for g in range(n // lanes):
        match_dst = [-1] * lanes
        def augment(src, seen):
            for dst in range(lanes):
                if buckets[src][dst] and not seen[dst]:
                    seen[dst] = True
                    if match_dst[dst] < 0 or augment(match_dst[dst], seen):
                        match_dst[dst] = src
                        match_src[src] = dst
                        return True
            return False
        for src in range(lanes):
            augment(src, [False] * lanes)
        for lane in range(lanes):
            dst_bank = match_src[lane]
            dst = buckets[lane][dst_bank].pop()
            gather_idx[g * lanes + lane] = perm[dst]
            scatter_idx[g * lanes + lane] = dst
    assert all(not b for row in buckets for b in row)
    return gather_idx, scatter_idx


_GATHER_IDX, _SCATTER_IDX = _conflict_free_schedule(_PERM)
_PACKED_IDX = (_GATHER_IDX | (_SCATTER_IDX << 11)).astype(np.int32)


def _make_kernel():
    mesh = plsc.VectorSubcoreMesh(core_axis_name="c", subcore_axis_name="s")

    @functools.partial(
        pl.kernel,
        mesh=mesh,
        out_type=jax.ShapeDtypeStruct((_BATCH, _DIM), jnp.float32),
        compiler_params=pltpu.CompilerParams(needs_layout_passes=False),
        scratch_types=[
            pltpu.VMEM((_DIM,), jnp.int32),
            pltpu.VMEM((_NBUF_IN, _CHUNK, _DIM), jnp.float32),
            pltpu.VMEM((_NBUF_OUT, _CHUNK, _DIM), jnp.float32),
        ] + [pltpu.SemaphoreType.DMA] * (_NBUF_IN + _NBUF_OUT),
    )
    def permute_kernel(sample_hbm, idx_hbm, out_hbm,
                       idx_v, in_v, out_v, *sems):
        num_cores = 2
        wid = lax.axis_index("s") * num_cores + lax.axis_index("c")
        row_base = wid * _ROWS_PER_WORKER
        sems_in = sems[:_NBUF_IN]
        sems_out = sems[_NBUF_IN:]

        def start_in(c):
            b = c % _NBUF_IN
            rows = row_base + c * _CHUNK
            return pltpu.async_copy(
                sample_hbm.at[pl.ds(rows, _CHUNK)], in_v.at[b], sems_in[b])

        def start_out(c):
            b = c % _NBUF_OUT
            rows = row_base + c * _CHUNK
            return pltpu.async_copy(
                out_v.at[b], out_hbm.at[pl.ds(rows, _CHUNK)], sems_out[b])

        def compute(bi, bo):
            @plsc.parallel_loop(0, _GROUPS, unroll=4)
            def _(g):
                off = pl.multiple_of(g * _LANES, _LANES)
                packed = idx_v[pl.ds(off, _LANES)]
                gidx = packed & 0x7FF
                sidx = lax.shift_right_logical(packed, 11)
                for r in range(_CHUNK):
                    sel_bi = jnp.full((_LANES,), bi, dtype=jnp.int32)
                    sel_bo = jnp.full((_LANES,), bo, dtype=jnp.int32)
                    sel_r = jnp.full((_LANES,), r, dtype=jnp.int32)
                    vals = plsc.load_gather(in_v, [sel_bi, sel_r, gidx])
                    plsc.store_scatter(out_v, [sel_bo, sel_r, sidx], vals)

        d_in = {c: start_in(c) for c in range(_NBUF_IN - 1)}
        pltpu.sync_copy(idx_hbm, idx_v)
        d_out = {}
        for c in range(_NUM_CHUNKS):
            if c + _NBUF_IN - 1 < _NUM_CHUNKS:
                d_in[c + _NBUF_IN - 1] = start_in(c + _NBUF_IN - 1)
            d_in[c].wait()
            if c >= _NBUF_OUT:
                d_out[c - _NBUF_OUT].wait()
            compute(c % _NBUF_IN, c % _NBUF_OUT)
            d_out[c] = start_out(c)
        for c in range(_NUM_CHUNKS - _NBUF_OUT, _NUM_CHUNKS):
            d_out[c].wait()

    return permute_kernel


_PERMUTE = _make_kernel()


def kernel(sample):
    return _PERMUTE(sample, jnp.asarray(_PACKED_IDX))


# speedup vs baseline: 1.0357x; 1.0357x over previous
import functools

import jax
import jax.numpy as jnp
import numpy as np
from jax import lax
from jax.experimental import pallas as pl
from jax.experimental.pallas import tpu as pltpu
from jax.experimental.pallas import tpu_sc as plsc

_DIM = 2048
_BATCH = 4096
_LANES = 16
_NUM_WORKERS = 32
_ROWS_PER_WORKER = _BATCH // _NUM_WORKERS
_CHUNK = 8
_NUM_CHUNKS = _ROWS_PER_WORKER // _CHUNK
_GROUPS = _DIM // _LANES
_NBUF_IN = 4
_NBUF_OUT = 3

_PERM = np.asarray(jax.random.permutation(jax.random.key(42), _DIM),
                   dtype=np.int32)


def _conflict_free_schedule(perm: np.ndarray):
    n, lanes = perm.shape[0], _LANES
    buckets = [[[] for _ in range(lanes)] for _ in range(lanes)]
    for dst in range(n):
        buckets[int(perm[dst]) % lanes][dst % lanes].append(dst)

    match_src = [0] * lanes
    gather_idx = np.empty(n, dtype=np.int32)
    scatter_idx = np.empty(n, dtype=np.int32)
    for g in range(n // lanes):
        match_dst = [-1] * lanes
        def augment(src, seen):
            for dst in range(lanes):
                if buckets[src][dst] and not seen[dst]:
                    seen[dst] = True
                    if match_dst[dst] < 0 or augment(match_dst[dst], seen):
                        match_dst[dst] = src
                        match_src[src] = dst
                        return True
            return False
        for src in range(lanes):
            augment(src, [False] * lanes)
        for lane in range(lanes):
            dst_bank = match_src[lane]
            dst = buckets[lane][dst_bank].pop()
            gather_idx[g * lanes + lane] = perm[dst]
            scatter_idx[g * lanes + lane] = dst
    assert all(not b for row in buckets for b in row)
    return gather_idx, scatter_idx


_GATHER_IDX, _SCATTER_IDX = _conflict_free_schedule(_PERM)
_PACKED_IDX = (_GATHER_IDX | (_SCATTER_IDX << 11)).astype(np.int32)


def _make_kernel():
    mesh = plsc.VectorSubcoreMesh(core_axis_name="c", subcore_axis_name="s")

    @functools.partial(
        pl.kernel,
        mesh=mesh,
        out_type=jax.ShapeDtypeStruct((_BATCH, _DIM), jnp.float32),
        compiler_params=pltpu.CompilerParams(needs_layout_passes=False),
        scratch_types=[
            pltpu.VMEM((_DIM,), jnp.int32),
            pltpu.VMEM((_NBUF_IN, _CHUNK, _DIM), jnp.float32),
            pltpu.VMEM((_NBUF_OUT, _CHUNK, _DIM), jnp.float32),
        ] + [pltpu.SemaphoreType.DMA] * (_NBUF_IN + _NBUF_OUT),
    )
    def permute_kernel(sample_hbm, idx_hbm, out_hbm,
                       idx_v, in_v, out_v, *sems):
        num_cores = 2
        wid = lax.axis_index("s") * num_cores + lax.axis_index("c")
        row_base = wid * _ROWS_PER_WORKER
        sems_in = sems[:_NBUF_IN]
        sems_out = sems[_NBUF_IN:]

        def start_in(c):
            b = c % _NBUF_IN
            rows = row_base + c * _CHUNK
            return pltpu.async_copy(
                sample_hbm.at[pl.ds(rows, _CHUNK)], in_v.at[b], sems_in[b])

        def start_out(c):
            b = c % _NBUF_OUT
            rows = row_base + c * _CHUNK
            return pltpu.async_copy(
                out_v.at[b], out_hbm.at[pl.ds(rows, _CHUNK)], sems_out[b])

        def compute(bi, bo):
            @plsc.parallel_loop(0, _GROUPS, unroll=2)
            def _(g):
                off = pl.multiple_of(g * _LANES, _LANES)
                packed = idx_v[pl.ds(off, _LANES)]
                gidx = packed & 0x7FF
                sidx = lax.shift_right_logical(packed, 11)
                for r in range(_CHUNK):
                    sel_bi = jnp.full((_LANES,), bi, dtype=jnp.int32)
                    sel_bo = jnp.full((_LANES,), bo, dtype=jnp.int32)
                    sel_r = jnp.full((_LANES,), r, dtype=jnp.int32)
                    vals = plsc.load_gather(in_v, [sel_bi, sel_r, gidx])
                    plsc.store_scatter(out_v, [sel_bo, sel_r, sidx], vals)

        d_in = {c: start_in(c) for c in range(_NBUF_IN - 1)}
        pltpu.sync_copy(idx_hbm, idx_v)
        d_out = {}
        for c in range(_NUM_CHUNKS):
            if c + _NBUF_IN - 1 < _NUM_CHUNKS:
                d_in[c + _NBUF_IN - 1] = start_in(c + _NBUF_IN - 1)
            d_in[c].wait()
            if c >= _NBUF_OUT:
                d_out[c - _NBUF_OUT].wait()
            compute(c % _NBUF_IN, c % _NBUF_OUT)
            d_out[c] = start_out(c)
        for c in range(_NUM_CHUNKS - _NBUF_OUT, _NUM_CHUNKS):
            d_out[c].wait()

    return permute_kernel


_PERMUTE = _make_kernel()


def kernel(sample):
    return _PERMUTE(sample, jnp.asarray(_PACKED_IDX))
